# baseline (speedup 1.0000x reference)
import functools

import jax
import jax.numpy as jnp
from jax import lax
from jax.experimental import pallas as pl
from jax.experimental.pallas import tpu as pltpu
from jax.experimental.pallas import tpu_sc as plsc

NC = 2
NS = 16
NW = NC * NS
CH = 5632
CH2 = 2816
NCH2 = 36

_mesh = plsc.VectorSubcoreMesh(core_axis_name="c", subcore_axis_name="s")


def _sc_degrees(n_pad, e_pad):
  nch = e_pad // (NW * CH)
  ew = e_pad // NW
  sl_sz = n_pad // NS

  @functools.partial(
      pl.kernel,
      mesh=_mesh,
      out_type=[
          jax.ShapeDtypeStruct((NC, n_pad), jnp.float32),
          jax.ShapeDtypeStruct((NC, n_pad), jnp.float32),
      ],
      scratch_types=[
          pltpu.VMEM((CH,), jnp.int32),
          pltpu.VMEM((CH,), jnp.int32),
          pltpu.VMEM((CH,), jnp.float32),
          pltpu.VMEM_SHARED((n_pad,), jnp.float32),
          pltpu.VMEM_SHARED((n_pad,), jnp.float32),
      ],
  )
  def deg_kernel(src_hbm, dst_hbm, zeros_hbm, ones_hbm,
                 dego_hbm, degi_hbm,
                 src_buf, dst_buf, ones_buf, dego_sh, degi_sh):
    c = lax.axis_index("c")
    s = lax.axis_index("s")
    sl = pl.ds(s * sl_sz, sl_sz)
    pltpu.sync_copy(zeros_hbm.at[sl], dego_sh.at[sl])
    pltpu.sync_copy(zeros_hbm.at[sl], degi_sh.at[sl])
    pltpu.sync_copy(ones_hbm, ones_buf)
    plsc.subcore_barrier()

    base = (c * NS + s) * ew

    def body(i, _):
      off = base + i * CH
      pltpu.sync_copy(src_hbm.at[pl.ds(off, CH)], src_buf)
      pltpu.sync_copy(dst_hbm.at[pl.ds(off, CH)], dst_buf)
      pltpu.sync_copy(ones_buf, dego_sh.at[src_buf], add=True)
      pltpu.sync_copy(ones_buf, degi_sh.at[dst_buf], add=True)
      return 0

    lax.fori_loop(0, nch, body, 0)
    plsc.subcore_barrier()
    pltpu.sync_copy(dego_sh.at[sl], dego_hbm.at[c].at[sl])
    pltpu.sync_copy(degi_sh.at[sl], degi_hbm.at[c].at[sl])

  return deg_kernel


def _sc_layer(n_pad, e_pad):
  nch = e_pad // (NW * CH2)
  ew = e_pad // NW
  sl_sz = n_pad // NS
  UNR = 8
  nvj = CH2 // (16 * UNR)
  assert nch % 6 == 0

  @functools.partial(
      pl.kernel,
      mesh=_mesh,
      out_type=jax.ShapeDtypeStruct((NC, n_pad), jnp.float32),
      scratch_types=[
          pltpu.VMEM((n_pad,), jnp.float32),
          [pltpu.VMEM((CH2,), jnp.int32)] * 2,
          [pltpu.VMEM((CH2,), jnp.int32)] * 3,
          [pltpu.VMEM((CH2,), jnp.float32)] * 3,
          pltpu.VMEM_SHARED((n_pad,), jnp.float32),
          [pltpu.SemaphoreType.DMA] * 2,
          [pltpu.SemaphoreType.DMA] * 3,
          [pltpu.SemaphoreType.DMA] * 3,
      ],
      compiler_params=pltpu.CompilerParams(needs_layout_passes=False),
  )
  def layer_kernel(u_hbm, src_hbm, dst_hbm, zeros_hbm,
                   acc_hbm,
                   u_tile, sB, dB, vB, acc_sh, sem_ls, sem_ld, sem_ss):
    c = lax.axis_index("c")
    s = lax.axis_index("s")
    sl = pl.ds(s * sl_sz, sl_sz)
    base = (c * NS + s) * ew

    pltpu.async_copy(src_hbm.at[pl.ds(base, CH2)], sB[0], sem_ls[0])
    pltpu.async_copy(dst_hbm.at[pl.ds(base, CH2)], dB[0], sem_ld[0])
    pltpu.sync_copy(zeros_hbm.at[sl], acc_sh.at[sl])
    pltpu.sync_copy(u_hbm, u_tile)
    plsc.subcore_barrier()

    def body(i, _):
      for t in range(6):
        sp, m = t % 2, t % 3
        spn, mn = (t + 1) % 2, (t + 1) % 3
        k = 6 * i + t
        off = base + k * CH2
        pltpu.make_async_copy(
            src_hbm.at[pl.ds(base, CH2)], sB[sp], sem_ls[sp]).wait()
        pltpu.make_async_copy(
            dst_hbm.at[pl.ds(base, CH2)], dB[m], sem_ld[m]).wait()

        @pl.when(k + 1 < nch)
        def _():
          pltpu.async_copy(
              src_hbm.at[pl.ds(off + CH2, CH2)], sB[spn], sem_ls[spn])
          pltpu.async_copy(
              dst_hbm.at[pl.ds(off + CH2, CH2)], dB[mn], sem_ld[mn])

        def gbody(j, _):
          for jj in range(UNR):
            o = (j * UNR + jj) * 16
            idx = sB[sp][pl.ds(o, 16)]
            vB[m][pl.ds(o, 16)] = plsc.load_gather(u_tile, [idx])
          return 0

        lax.fori_loop(0, nvj, gbody, 0)
        pltpu.sync_copy(vB[m], acc_sh.at[dB[m]], add=True)
      return 0

    lax.fori_loop(0, nch // 6, body, 0)
    plsc.subcore_barrier()
    pltpu.sync_copy(acc_sh.at[sl], acc_hbm.at[c].at[sl])

  return layer_kernel


def _tc_norms(r):
  def body(dego_ref, degi_ref, h_ref, ns_ref, nd_ref, u0_ref):
    od = dego_ref[0] + dego_ref[1]
    idg = degi_ref[0] + degi_ref[1]
    ns = lax.rsqrt(jnp.maximum(od, 1.0))
    nd = lax.rsqrt(jnp.maximum(idg, 1.0))
    ns_ref[...] = ns
    nd_ref[...] = nd
    u0_ref[...] = h_ref[...] * ns

  return pl.pallas_call(
      body,
      out_shape=[
          jax.ShapeDtypeStruct((r, 128), jnp.float32),
          jax.ShapeDtypeStruct((r, 128), jnp.float32),
          jax.ShapeDtypeStruct((r, 128), jnp.float32),
      ],
  )


def _tc_merge(r):
  def body(acc_ref, ns_ref, nd_ref, u_ref, c5_ref):
    v = (acc_ref[0] + acc_ref[1]) * nd_ref[...]
    u_ref[...] = v * ns_ref[...]
    c5_ref[...] = jnp.sum(v * v).reshape(1, 1)

  return pl.pallas_call(
      body,
      out_shape=[
          jax.ShapeDtypeStruct((r, 128), jnp.float32),
          jax.ShapeDtypeStruct((1, 1), jnp.float32),
      ],
  )


def kernel(h, edge_index):
  n = h.shape[0]
  e = edge_index.shape[1]
  n_pad = ((n + 2047) // 2048) * 2048
  r = n_pad // 128
  ew_quant = 6 * CH2
  e_pad = NW * (((e + NW * ew_quant - 1) // (NW * ew_quant)) * ew_quant)

  src = edge_index[0].astype(jnp.int32)
  dst = edge_index[1].astype(jnp.int32)
  if e_pad != e:
    pad = jnp.full((e_pad - e,), n_pad - 1, dtype=jnp.int32)
    src = jnp.concatenate([src, pad])
    dst = jnp.concatenate([dst, pad])
  hv = jnp.pad(h[:, 0], (0, n_pad - n))
  zeros = jnp.zeros((n_pad,), jnp.float32)
  ones = jnp.ones((CH,), jnp.float32)

  dego, degi = _sc_degrees(n_pad, e_pad)(src, dst, zeros, ones)
  ns, nd, u = _tc_norms(r)(
      dego.reshape(NC, r, 128), degi.reshape(NC, r, 128), hv.reshape(r, 128)
  )

  layer = _sc_layer(n_pad, e_pad)
  merge = _tc_merge(r)
  c5s = []
  for _ in range(3):
    accp = layer(u.reshape(-1), src, dst, zeros)
    u, c5 = merge(accp.reshape(NC, r, 128), ns, nd)
    c5s.append(c5[0, 0])
  return jnp.stack(c5s)


# speedup vs baseline: 1.0906x; 1.0906x over previous
import functools

import jax
import jax.numpy as jnp
from jax import lax
from jax.experimental import pallas as pl
from jax.experimental.pallas import tpu as pltpu
from jax.experimental.pallas import tpu_sc as plsc

NC = 2
NS = 16
NW = NC * NS
CH = 5632
CH2 = 2816
NCH2 = 36

_mesh = plsc.VectorSubcoreMesh(core_axis_name="c", subcore_axis_name="s")


def _sc_degrees(n_pad, e_pad):
  nch = e_pad // (NW * CH)
  ew = e_pad // NW
  sl_sz = n_pad // NS

  @functools.partial(
      pl.kernel,
      mesh=_mesh,
      out_type=[
          jax.ShapeDtypeStruct((NC, n_pad), jnp.float32),
          jax.ShapeDtypeStruct((NC, n_pad), jnp.float32),
      ],
      scratch_types=[
          pltpu.VMEM((CH,), jnp.int32),
          pltpu.VMEM((CH,), jnp.int32),
          pltpu.VMEM((CH,), jnp.float32),
          pltpu.VMEM_SHARED((n_pad,), jnp.float32),
          pltpu.VMEM_SHARED((n_pad,), jnp.float32),
      ],
  )
  def deg_kernel(src_hbm, dst_hbm, zeros_hbm, ones_hbm,
                 dego_hbm, degi_hbm,
                 src_buf, dst_buf, ones_buf, dego_sh, degi_sh):
    c = lax.axis_index("c")
    s = lax.axis_index("s")
    sl = pl.ds(s * sl_sz, sl_sz)
    pltpu.sync_copy(zeros_hbm.at[sl], dego_sh.at[sl])
    pltpu.sync_copy(zeros_hbm.at[sl], degi_sh.at[sl])
    pltpu.sync_copy(ones_hbm, ones_buf)
    plsc.subcore_barrier()

    base = (c * NS + s) * ew

    def body(i, _):
      off = base + i * CH
      pltpu.sync_copy(src_hbm.at[pl.ds(off, CH)], src_buf)
      pltpu.sync_copy(dst_hbm.at[pl.ds(off, CH)], dst_buf)
      pltpu.sync_copy(ones_buf, dego_sh.at[src_buf], add=True)
      pltpu.sync_copy(ones_buf, degi_sh.at[dst_buf], add=True)
      return 0

    lax.fori_loop(0, nch, body, 0)
    plsc.subcore_barrier()
    pltpu.sync_copy(dego_sh.at[sl], dego_hbm.at[c].at[sl])
    pltpu.sync_copy(degi_sh.at[sl], degi_hbm.at[c].at[sl])

  return deg_kernel


def _sc_layer(n_pad, e_pad):
  nch = e_pad // (NW * CH2)
  ew = e_pad // NW
  sl_sz = n_pad // NS
  UNR = 8
  nvj = CH2 // (16 * UNR)
  assert nch % 6 == 0

  @functools.partial(
      pl.kernel,
      mesh=_mesh,
      out_type=jax.ShapeDtypeStruct((NC, n_pad), jnp.float32),
      scratch_types=[
          pltpu.VMEM((n_pad,), jnp.float32),
          [pltpu.VMEM((CH2,), jnp.int32)] * 2,
          [pltpu.VMEM((CH2,), jnp.int32)] * 3,
          [pltpu.VMEM((CH2,), jnp.float32)] * 3,
          pltpu.VMEM_SHARED((n_pad,), jnp.float32),
          [pltpu.SemaphoreType.DMA] * 2,
          [pltpu.SemaphoreType.DMA] * 3,
          [pltpu.SemaphoreType.DMA] * 3,
      ],
      compiler_params=pltpu.CompilerParams(needs_layout_passes=False),
  )
  def layer_kernel(u_hbm, src_hbm, dst_hbm, zeros_hbm,
                   acc_hbm,
                   u_tile, sB, dB, vB, acc_sh, sem_ls, sem_ld, sem_ss):
    c = lax.axis_index("c")
    s = lax.axis_index("s")
    sl = pl.ds(s * sl_sz, sl_sz)
    base = (c * NS + s) * ew

    pltpu.async_copy(src_hbm.at[pl.ds(base, CH2)], sB[0], sem_ls[0])
    pltpu.async_copy(dst_hbm.at[pl.ds(base, CH2)], dB[0], sem_ld[0])
    pltpu.sync_copy(zeros_hbm.at[sl], acc_sh.at[sl])
    pltpu.sync_copy(u_hbm, u_tile)
    plsc.subcore_barrier()

    def body(i, _):
      for t in range(6):
        sp, m = t % 2, t % 3
        spn, mn = (t + 1) % 2, (t + 1) % 3
        k = 6 * i + t
        off = base + k * CH2
        pltpu.make_async_copy(
            src_hbm.at[pl.ds(base, CH2)], sB[sp], sem_ls[sp]).wait()
        pltpu.make_async_copy(
            dst_hbm.at[pl.ds(base, CH2)], dB[m], sem_ld[m]).wait()

        @pl.when(k + 1 < nch)
        def _():
          pltpu.async_copy(
              src_hbm.at[pl.ds(off + CH2, CH2)], sB[spn], sem_ls[spn])
          pltpu.async_copy(
              dst_hbm.at[pl.ds(off + CH2, CH2)], dB[mn], sem_ld[mn])

        @plsc.parallel_loop(0, CH2 // 16, 1, unroll=UNR)
        def _(j):
          o = j * 16
          idx = sB[sp][pl.ds(o, 16)]
          vB[m][pl.ds(o, 16)] = plsc.load_gather(u_tile, [idx])
        pltpu.sync_copy(vB[m], acc_sh.at[dB[m]], add=True)
      return 0

    lax.fori_loop(0, nch // 6, body, 0)
    plsc.subcore_barrier()
    pltpu.sync_copy(acc_sh.at[sl], acc_hbm.at[c].at[sl])

  return layer_kernel


def _tc_norms(r):
  def body(dego_ref, degi_ref, h_ref, ns_ref, nd_ref, u0_ref):
    od = dego_ref[0] + dego_ref[1]
    idg = degi_ref[0] + degi_ref[1]
    ns = lax.rsqrt(jnp.maximum(od, 1.0))
    nd = lax.rsqrt(jnp.maximum(idg, 1.0))
    ns_ref[...] = ns
    nd_ref[...] = nd
    u0_ref[...] = h_ref[...] * ns

  return pl.pallas_call(
      body,
      out_shape=[
          jax.ShapeDtypeStruct((r, 128), jnp.float32),
          jax.ShapeDtypeStruct((r, 128), jnp.float32),
          jax.ShapeDtypeStruct((r, 128), jnp.float32),
      ],
  )


def _tc_merge(r):
  def body(acc_ref, ns_ref, nd_ref, u_ref, c5_ref):
    v = (acc_ref[0] + acc_ref[1]) * nd_ref[...]
    u_ref[...] = v * ns_ref[...]
    c5_ref[...] = jnp.sum(v * v).reshape(1, 1)

  return pl.pallas_call(
      body,
      out_shape=[
          jax.ShapeDtypeStruct((r, 128), jnp.float32),
          jax.ShapeDtypeStruct((1, 1), jnp.float32),
      ],
  )


def kernel(h, edge_index):
  n = h.shape[0]
  e = edge_index.shape[1]
  n_pad = ((n + 2047) // 2048) * 2048
  r = n_pad // 128
  ew_quant = 6 * CH2
  e_pad = NW * (((e + NW * ew_quant - 1) // (NW * ew_quant)) * ew_quant)

  src = edge_index[0].astype(jnp.int32)
  dst = edge_index[1].astype(jnp.int32)
  if e_pad != e:
    pad = jnp.full((e_pad - e,), n_pad - 1, dtype=jnp.int32)
    src = jnp.concatenate([src, pad])
    dst = jnp.concatenate([dst, pad])
  hv = jnp.pad(h[:, 0], (0, n_pad - n))
  zeros = jnp.zeros((n_pad,), jnp.float32)
  ones = jnp.ones((CH,), jnp.float32)

  dego, degi = _sc_degrees(n_pad, e_pad)(src, dst, zeros, ones)
  ns, nd, u = _tc_norms(r)(
      dego.reshape(NC, r, 128), degi.reshape(NC, r, 128), hv.reshape(r, 128)
  )

  layer = _sc_layer(n_pad, e_pad)
  merge = _tc_merge(r)
  c5s = []
  for _ in range(3):
    accp = layer(u.reshape(-1), src, dst, zeros)
    u, c5 = merge(accp.reshape(NC, r, 128), ns, nd)
    c5s.append(c5[0, 0])
  return jnp.stack(c5s)


# speedup vs baseline: 1.6996x; 1.5584x over previous
import functools

import jax
import jax.numpy as jnp
from jax import lax
from jax.experimental import pallas as pl
from jax.experimental.pallas import tpu as pltpu
from jax.experimental.pallas import tpu_sc as plsc

NC = 2
NS = 16
NW = NC * NS
CH = 2944
CH2 = 2944

_mesh = plsc.VectorSubcoreMesh(core_axis_name="c", subcore_axis_name="s")


def _sc_degrees(n_pad, e_pad):
  nch = e_pad // (NW * CH)
  ew = e_pad // NW
  sl_sz = n_pad // NS

  @functools.partial(
      pl.kernel,
      mesh=_mesh,
      out_type=[
          jax.ShapeDtypeStruct((NC, n_pad), jnp.float32),
          jax.ShapeDtypeStruct((NC, n_pad), jnp.float32),
      ],
      scratch_types=[
          pltpu.VMEM((CH,), jnp.int32),
          pltpu.VMEM((CH,), jnp.int32),
          pltpu.VMEM((CH,), jnp.float32),
          pltpu.VMEM_SHARED((n_pad,), jnp.float32),
          pltpu.VMEM_SHARED((n_pad,), jnp.float32),
      ],
  )
  def deg_kernel(src_hbm, dst_hbm, zeros_hbm, ones_hbm,
                 dego_hbm, degi_hbm,
                 src_buf, dst_buf, ones_buf, dego_sh, degi_sh):
    c = lax.axis_index("c")
    s = lax.axis_index("s")
    sl = pl.ds(s * sl_sz, sl_sz)
    pltpu.sync_copy(zeros_hbm.at[sl], dego_sh.at[sl])
    pltpu.sync_copy(zeros_hbm.at[sl], degi_sh.at[sl])
    pltpu.sync_copy(ones_hbm, ones_buf)
    plsc.subcore_barrier()

    base = (c * NS + s) * ew

    def body(i, _):
      off = base + i * CH
      pltpu.sync_copy(src_hbm.at[pl.ds(off, CH)], src_buf)
      pltpu.sync_copy(dst_hbm.at[pl.ds(off, CH)], dst_buf)
      pltpu.sync_copy(ones_buf, dego_sh.at[src_buf], add=True)
      pltpu.sync_copy(ones_buf, degi_sh.at[dst_buf], add=True)
      return 0

    lax.fori_loop(0, nch, body, 0)
    plsc.subcore_barrier()
    pltpu.sync_copy(dego_sh.at[sl], dego_hbm.at[c].at[sl])
    pltpu.sync_copy(degi_sh.at[sl], degi_hbm.at[c].at[sl])

  return deg_kernel


def _sc_layer(n_pad, e_pad):
  nch = e_pad // (NW * CH2)
  ew = e_pad // NW
  sl_sz = n_pad // NS
  UNR = 8

  @functools.partial(
      pl.kernel,
      mesh=_mesh,
      out_type=jax.ShapeDtypeStruct((NC, n_pad), jnp.float32),
      scratch_types=[
          pltpu.VMEM((n_pad,), jnp.float32),
          [pltpu.VMEM((CH2,), jnp.int32)] * 2,
          [pltpu.VMEM((CH2,), jnp.int32)] * 3,
          [pltpu.VMEM((CH2,), jnp.float32)] * 3,
          pltpu.VMEM_SHARED((n_pad,), jnp.float32),
          [pltpu.SemaphoreType.DMA] * 2,
          [pltpu.SemaphoreType.DMA] * 3,
          [pltpu.SemaphoreType.DMA] * 3,
      ],
      compiler_params=pltpu.CompilerParams(needs_layout_passes=False),
  )
  def layer_kernel(u_hbm, src_hbm, dst_hbm, zeros_hbm,
                   acc_hbm,
                   u_tile, sB, dB, vB, acc_sh, sem_ls, sem_ld, sem_ss):
    c = lax.axis_index("c")
    s = lax.axis_index("s")
    sl = pl.ds(s * sl_sz, sl_sz)
    base = (c * NS + s) * ew

    loads = [None, None]
    dloads = [None, None, None]
    scats = [None, None, None]
    loads[0] = pltpu.async_copy(
        src_hbm.at[pl.ds(base, CH2)], sB[0], sem_ls[0])
    dloads[0] = pltpu.async_copy(
        dst_hbm.at[pl.ds(base, CH2)], dB[0], sem_ld[0])
    pltpu.sync_copy(zeros_hbm.at[sl], acc_sh.at[sl])
    pltpu.sync_copy(u_hbm, u_tile)
    plsc.subcore_barrier()

    for k in range(nch):
      sp, m = k % 2, k % 3
      spn, mn = (k + 1) % 2, (k + 1) % 3
      off = base + k * CH2
      loads[sp].wait()
      dloads[m].wait()

      if k + 1 < nch:
        loads[spn] = pltpu.async_copy(
            src_hbm.at[pl.ds(off + CH2, CH2)], sB[spn], sem_ls[spn])
        dloads[mn] = pltpu.async_copy(
            dst_hbm.at[pl.ds(off + CH2, CH2)], dB[mn], sem_ld[mn])

      @plsc.parallel_loop(0, CH2 // 16, 1, unroll=UNR)
      def _(j):
        o = j * 16
        idx = sB[sp][pl.ds(o, 16)]
        vB[m][pl.ds(o, 16)] = plsc.load_gather(u_tile, [idx])

      mp = (k - 1) % 3
      if scats[mp] is not None:
        scats[mp].wait()
        scats[mp] = None
      scats[m] = pltpu.async_copy(vB[m], acc_sh.at[dB[m]], sem_ss[m], add=True)

    scats[(nch - 1) % 3].wait()
    plsc.subcore_barrier()
    pltpu.sync_copy(acc_sh.at[sl], acc_hbm.at[c].at[sl])

  return layer_kernel


def _tc_norms(r):
  def body(dego_ref, degi_ref, h_ref, ns_ref, nd_ref, u0_ref):
    od = dego_ref[0] + dego_ref[1]
    idg = degi_ref[0] + degi_ref[1]
    ns = lax.rsqrt(jnp.maximum(od, 1.0))
    nd = lax.rsqrt(jnp.maximum(idg, 1.0))
    ns_ref[...] = ns
    nd_ref[...] = nd
    u0_ref[...] = h_ref[...] * ns

  return pl.pallas_call(
      body,
      out_shape=[
          jax.ShapeDtypeStruct((r, 128), jnp.float32),
          jax.ShapeDtypeStruct((r, 128), jnp.float32),
          jax.ShapeDtypeStruct((r, 128), jnp.float32),
      ],
  )


def _tc_merge(r):
  def body(acc_ref, ns_ref, nd_ref, u_ref, c5_ref):
    v = (acc_ref[0] + acc_ref[1]) * nd_ref[...]
    u_ref[...] = v * ns_ref[...]
    c5_ref[...] = jnp.sum(v * v).reshape(1, 1)

  return pl.pallas_call(
      body,
      out_shape=[
          jax.ShapeDtypeStruct((r, 128), jnp.float32),
          jax.ShapeDtypeStruct((1, 1), jnp.float32),
      ],
  )


def kernel(h, edge_index):
  n = h.shape[0]
  e = edge_index.shape[1]
  n_pad = ((n + 2047) // 2048) * 2048
  r = n_pad // 128
  e_pad = NW * (((e + NW * CH2 - 1) // (NW * CH2)) * CH2)

  src = edge_index[0].astype(jnp.int32)
  dst = edge_index[1].astype(jnp.int32)
  if e_pad != e:
    pad = jnp.full((e_pad - e,), n_pad - 1, dtype=jnp.int32)
    src = jnp.concatenate([src, pad])
    dst = jnp.concatenate([dst, pad])
  hv = jnp.pad(h[:, 0], (0, n_pad - n))
  zeros = jnp.zeros((n_pad,), jnp.float32)
  ones = jnp.ones((CH,), jnp.float32)

  dego, degi = _sc_degrees(n_pad, e_pad)(src, dst, zeros, ones)
  ns, nd, u = _tc_norms(r)(
      dego.reshape(NC, r, 128), degi.reshape(NC, r, 128), hv.reshape(r, 128)
  )

  layer = _sc_layer(n_pad, e_pad)
  merge = _tc_merge(r)
  c5s = []
  for _ in range(3):
    accp = layer(u.reshape(-1), src, dst, zeros)
    u, c5 = merge(accp.reshape(NC, r, 128), ns, nd)
    c5s.append(c5[0, 0])
  return jnp.stack(c5s)


# speedup vs baseline: 1.9589x; 1.1525x over previous
import functools

import jax
import jax.numpy as jnp
from jax import lax
from jax.experimental import pallas as pl
from jax.experimental.pallas import tpu as pltpu
from jax.experimental.pallas import tpu_sc as plsc

NC = 2
NS = 16
NW = NC * NS
CH = 2944
CH2 = 2944

_mesh = plsc.VectorSubcoreMesh(core_axis_name="c", subcore_axis_name="s")


def _sc_degrees(n_pad, e_pad):
  nch = e_pad // (NW * CH)
  ew = e_pad // NW
  sl_sz = n_pad // NS

  @functools.partial(
      pl.kernel,
      mesh=_mesh,
      out_type=[
          jax.ShapeDtypeStruct((NC, n_pad), jnp.float32),
          jax.ShapeDtypeStruct((NC, n_pad), jnp.float32),
      ],
      scratch_types=[
          [pltpu.VMEM((CH,), jnp.int32)] * 3,
          [pltpu.VMEM((CH,), jnp.int32)] * 3,
          pltpu.VMEM((CH,), jnp.float32),
          pltpu.VMEM_SHARED((n_pad,), jnp.float32),
          pltpu.VMEM_SHARED((n_pad,), jnp.float32),
          [pltpu.SemaphoreType.DMA] * 3,
          [pltpu.SemaphoreType.DMA] * 3,
          [pltpu.SemaphoreType.DMA] * 3,
          [pltpu.SemaphoreType.DMA] * 3,
      ],
  )
  def deg_kernel(src_hbm, dst_hbm, zeros_hbm, ones_hbm,
                 dego_hbm, degi_hbm,
                 sB, dB, ones_buf, dego_sh, degi_sh,
                 sem_ls, sem_ld, sem_so, sem_si):
    c = lax.axis_index("c")
    s = lax.axis_index("s")
    sl = pl.ds(s * sl_sz, sl_sz)
    base = (c * NS + s) * ew

    loads = [None] * 3
    dloads = [None] * 3
    so = [None] * 3
    si = [None] * 3
    loads[0] = pltpu.async_copy(
        src_hbm.at[pl.ds(base, CH)], sB[0], sem_ls[0])
    dloads[0] = pltpu.async_copy(
        dst_hbm.at[pl.ds(base, CH)], dB[0], sem_ld[0])
    pltpu.sync_copy(zeros_hbm.at[sl], dego_sh.at[sl])
    pltpu.sync_copy(zeros_hbm.at[sl], degi_sh.at[sl])
    pltpu.sync_copy(ones_hbm, ones_buf)
    plsc.subcore_barrier()

    for k in range(nch):
      m, mn = k % 3, (k + 1) % 3
      off = base + k * CH
      loads[m].wait()
      dloads[m].wait()
      if k + 1 < nch:
        if so[mn] is not None:
          so[mn].wait()
          so[mn] = None
        if si[mn] is not None:
          si[mn].wait()
          si[mn] = None
        loads[mn] = pltpu.async_copy(
            src_hbm.at[pl.ds(off + CH, CH)], sB[mn], sem_ls[mn])
        dloads[mn] = pltpu.async_copy(
            dst_hbm.at[pl.ds(off + CH, CH)], dB[mn], sem_ld[mn])
      so[m] = pltpu.async_copy(
          ones_buf, dego_sh.at[sB[m]], sem_so[m], add=True)
      si[m] = pltpu.async_copy(
          ones_buf, degi_sh.at[dB[m]], sem_si[m], add=True)

    for m in range(3):
      if so[m] is not None:
        so[m].wait()
      if si[m] is not None:
        si[m].wait()
    plsc.subcore_barrier()
    pltpu.sync_copy(dego_sh.at[sl], dego_hbm.at[c].at[sl])
    pltpu.sync_copy(degi_sh.at[sl], degi_hbm.at[c].at[sl])

  return deg_kernel


def _sc_layer(n_pad, e_pad):
  nch = e_pad // (NW * CH2)
  ew = e_pad // NW
  sl_sz = n_pad // NS
  UNR = 8

  @functools.partial(
      pl.kernel,
      mesh=_mesh,
      out_type=jax.ShapeDtypeStruct((NC, n_pad), jnp.float32),
      scratch_types=[
          pltpu.VMEM((n_pad,), jnp.float32),
          [pltpu.VMEM((CH2,), jnp.int32)] * 2,
          [pltpu.VMEM((CH2,), jnp.int32)] * 3,
          [pltpu.VMEM((CH2,), jnp.float32)] * 3,
          pltpu.VMEM_SHARED((n_pad,), jnp.float32),
          [pltpu.SemaphoreType.DMA] * 2,
          [pltpu.SemaphoreType.DMA] * 3,
          [pltpu.SemaphoreType.DMA] * 3,
      ],
      compiler_params=pltpu.CompilerParams(needs_layout_passes=False),
  )
  def layer_kernel(u_hbm, src_hbm, dst_hbm, zeros_hbm,
                   acc_hbm,
                   u_tile, sB, dB, vB, acc_sh, sem_ls, sem_ld, sem_ss):
    c = lax.axis_index("c")
    s = lax.axis_index("s")
    sl = pl.ds(s * sl_sz, sl_sz)
    base = (c * NS + s) * ew

    loads = [None, None]
    dloads = [None, None, None]
    scats = [None, None, None]
    loads[0] = pltpu.async_copy(
        src_hbm.at[pl.ds(base, CH2)], sB[0], sem_ls[0])
    dloads[0] = pltpu.async_copy(
        dst_hbm.at[pl.ds(base, CH2)], dB[0], sem_ld[0])
    pltpu.sync_copy(zeros_hbm.at[sl], acc_sh.at[sl])
    pltpu.sync_copy(u_hbm, u_tile)
    plsc.subcore_barrier()

    for k in range(nch):
      sp, m = k % 2, k % 3
      spn, mn = (k + 1) % 2, (k + 1) % 3
      off = base + k * CH2
      loads[sp].wait()
      dloads[m].wait()

      if k + 1 < nch:
        loads[spn] = pltpu.async_copy(
            src_hbm.at[pl.ds(off + CH2, CH2)], sB[spn], sem_ls[spn])
        dloads[mn] = pltpu.async_copy(
            dst_hbm.at[pl.ds(off + CH2, CH2)], dB[mn], sem_ld[mn])

      @plsc.parallel_loop(0, CH2 // 16, 1, unroll=UNR)
      def _(j):
        o = j * 16
        idx = sB[sp][pl.ds(o, 16)]
        vB[m][pl.ds(o, 16)] = plsc.load_gather(u_tile, [idx])

      mp = (k - 1) % 3
      if scats[mp] is not None:
        scats[mp].wait()
        scats[mp] = None
      scats[m] = pltpu.async_copy(vB[m], acc_sh.at[dB[m]], sem_ss[m], add=True)

    scats[(nch - 1) % 3].wait()
    plsc.subcore_barrier()
    pltpu.sync_copy(acc_sh.at[sl], acc_hbm.at[c].at[sl])

  return layer_kernel


def _tc_norms(r):
  def body(dego_ref, degi_ref, h_ref, ns_ref, nd_ref, u0_ref):
    od = dego_ref[0] + dego_ref[1]
    idg = degi_ref[0] + degi_ref[1]
    ns = lax.rsqrt(jnp.maximum(od, 1.0))
    nd = lax.rsqrt(jnp.maximum(idg, 1.0))
    ns_ref[...] = ns
    nd_ref[...] = nd
    u0_ref[...] = h_ref[...] * ns

  return pl.pallas_call(
      body,
      out_shape=[
          jax.ShapeDtypeStruct((r, 128), jnp.float32),
          jax.ShapeDtypeStruct((r, 128), jnp.float32),
          jax.ShapeDtypeStruct((r, 128), jnp.float32),
      ],
  )


def _tc_merge(r):
  def body(acc_ref, ns_ref, nd_ref, u_ref, c5_ref):
    v = (acc_ref[0] + acc_ref[1]) * nd_ref[...]
    u_ref[...] = v * ns_ref[...]
    c5_ref[...] = jnp.sum(v * v).reshape(1, 1)

  return pl.pallas_call(
      body,
      out_shape=[
          jax.ShapeDtypeStruct((r, 128), jnp.float32),
          jax.ShapeDtypeStruct((1, 1), jnp.float32),
      ],
  )


def kernel(h, edge_index):
  n = h.shape[0]
  e = edge_index.shape[1]
  n_pad = ((n + 2047) // 2048) * 2048
  r = n_pad // 128
  e_pad = NW * (((e + NW * CH2 - 1) // (NW * CH2)) * CH2)

  src = edge_index[0].astype(jnp.int32)
  dst = edge_index[1].astype(jnp.int32)
  if e_pad != e:
    pad = jnp.full((e_pad - e,), n_pad - 1, dtype=jnp.int32)
    src = jnp.concatenate([src, pad])
    dst = jnp.concatenate([dst, pad])
  hv = jnp.pad(h[:, 0], (0, n_pad - n))
  zeros = jnp.zeros((n_pad,), jnp.float32)
  ones = jnp.ones((CH,), jnp.float32)

  dego, degi = _sc_degrees(n_pad, e_pad)(src, dst, zeros, ones)
  ns, nd, u = _tc_norms(r)(
      dego.reshape(NC, r, 128), degi.reshape(NC, r, 128), hv.reshape(r, 128)
  )

  layer = _sc_layer(n_pad, e_pad)
  merge = _tc_merge(r)
  c5s = []
  for _ in range(3):
    accp = layer(u.reshape(-1), src, dst, zeros)
    u, c5 = merge(accp.reshape(NC, r, 128), ns, nd)
    c5s.append(c5[0, 0])
  return jnp.stack(c5s)


# speedup vs baseline: 1.9967x; 1.0193x over previous
import functools

import jax
import jax.numpy as jnp
from jax import lax
from jax.experimental import pallas as pl
from jax.experimental.pallas import tpu as pltpu
from jax.experimental.pallas import tpu_sc as plsc

NC = 2
NS = 16
NW = NC * NS
CH = 2944
CH2 = 2944

_mesh = plsc.VectorSubcoreMesh(core_axis_name="c", subcore_axis_name="s")


def _sc_degrees(n_pad, e_pad):
  nch = e_pad // (NW * CH)
  ew = e_pad // NW
  sl_sz = n_pad // NS

  @functools.partial(
      pl.kernel,
      mesh=_mesh,
      out_type=[
          jax.ShapeDtypeStruct((NC, n_pad), jnp.float32),
          jax.ShapeDtypeStruct((NC, n_pad), jnp.float32),
      ],
      scratch_types=[
          [pltpu.VMEM((CH,), jnp.int32)] * 3,
          [pltpu.VMEM((CH,), jnp.int32)] * 3,
          pltpu.VMEM((CH,), jnp.float32),
          pltpu.VMEM_SHARED((n_pad,), jnp.float32),
          pltpu.VMEM_SHARED((n_pad,), jnp.float32),
          [pltpu.SemaphoreType.DMA] * 3,
          [pltpu.SemaphoreType.DMA] * 3,
          [pltpu.SemaphoreType.DMA] * 3,
          [pltpu.SemaphoreType.DMA] * 3,
      ],
  )
  def deg_kernel(src_hbm, dst_hbm, zeros_hbm, ones_hbm,
                 dego_hbm, degi_hbm,
                 sB, dB, ones_buf, dego_sh, degi_sh,
                 sem_ls, sem_ld, sem_so, sem_si):
    c = lax.axis_index("c")
    s = lax.axis_index("s")
    sl = pl.ds(s * sl_sz, sl_sz)
    base = (c * NS + s) * ew

    loads = [None] * 3
    dloads = [None] * 3
    so = [None] * 3
    si = [None] * 3
    loads[0] = pltpu.async_copy(
        src_hbm.at[pl.ds(base, CH)], sB[0], sem_ls[0])
    dloads[0] = pltpu.async_copy(
        dst_hbm.at[pl.ds(base, CH)], dB[0], sem_ld[0])
    pltpu.sync_copy(zeros_hbm.at[sl], dego_sh.at[sl])
    pltpu.sync_copy(zeros_hbm.at[sl], degi_sh.at[sl])
    pltpu.sync_copy(ones_hbm, ones_buf)
    plsc.subcore_barrier()

    for k in range(nch):
      m, mn = k % 3, (k + 1) % 3
      off = base + k * CH
      loads[m].wait()
      dloads[m].wait()
      if k + 1 < nch:
        if so[mn] is not None:
          so[mn].wait()
          so[mn] = None
        if si[mn] is not None:
          si[mn].wait()
          si[mn] = None
        loads[mn] = pltpu.async_copy(
            src_hbm.at[pl.ds(off + CH, CH)], sB[mn], sem_ls[mn])
        dloads[mn] = pltpu.async_copy(
            dst_hbm.at[pl.ds(off + CH, CH)], dB[mn], sem_ld[mn])
      so[m] = pltpu.async_copy(
          ones_buf, dego_sh.at[sB[m]], sem_so[m], add=True)
      si[m] = pltpu.async_copy(
          ones_buf, degi_sh.at[dB[m]], sem_si[m], add=True)

    for m in range(3):
      if so[m] is not None:
        so[m].wait()
      if si[m] is not None:
        si[m].wait()
    plsc.subcore_barrier()
    pltpu.sync_copy(dego_sh.at[sl], dego_hbm.at[c].at[sl])
    pltpu.sync_copy(degi_sh.at[sl], degi_hbm.at[c].at[sl])

  return deg_kernel


def _sc_layer(n_pad, e_pad):
  nch = e_pad // (NW * CH2)
  ew = e_pad // NW
  sl_sz = n_pad // NS
  UNR = 8

  @functools.partial(
      pl.kernel,
      mesh=_mesh,
      out_type=jax.ShapeDtypeStruct((NC, n_pad), jnp.float32),
      scratch_types=[
          pltpu.VMEM((n_pad,), jnp.float32),
          [pltpu.VMEM((CH2,), jnp.int32)] * 2,
          [pltpu.VMEM((CH2,), jnp.int32)] * 3,
          [pltpu.VMEM((CH2,), jnp.float32)] * 3,
          pltpu.VMEM_SHARED((n_pad,), jnp.float32),
          [pltpu.SemaphoreType.DMA] * 2,
          [pltpu.SemaphoreType.DMA] * 3,
          [pltpu.SemaphoreType.DMA] * 3,
      ],
      compiler_params=pltpu.CompilerParams(needs_layout_passes=False),
  )
  def layer_kernel(u_hbm, src_hbm, dst_hbm, zeros_hbm,
                   acc_hbm,
                   u_tile, sB, dB, vB, acc_sh, sem_ls, sem_ld, sem_ss):
    c = lax.axis_index("c")
    s = lax.axis_index("s")
    sl = pl.ds(s * sl_sz, sl_sz)
    base = (c * NS + s) * ew

    loads = [None, None]
    dloads = [None, None, None]
    scats = [None, None, None]
    loads[0] = pltpu.async_copy(
        src_hbm.at[pl.ds(base, CH2)], sB[0], sem_ls[0])
    dloads[0] = pltpu.async_copy(
        dst_hbm.at[pl.ds(base, CH2)], dB[0], sem_ld[0])
    pltpu.sync_copy(zeros_hbm.at[sl], acc_sh.at[sl])
    pltpu.sync_copy(u_hbm, u_tile)
    plsc.subcore_barrier()

    for k in range(nch):
      sp, m = k % 2, k % 3
      spn, mn = (k + 1) % 2, (k + 1) % 3
      off = base + k * CH2
      loads[sp].wait()
      dloads[m].wait()

      if k + 1 < nch:
        if scats[mn] is not None:
          scats[mn].wait()
          scats[mn] = None
        loads[spn] = pltpu.async_copy(
            src_hbm.at[pl.ds(off + CH2, CH2)], sB[spn], sem_ls[spn])
        dloads[mn] = pltpu.async_copy(
            dst_hbm.at[pl.ds(off + CH2, CH2)], dB[mn], sem_ld[mn])

      @plsc.parallel_loop(0, CH2 // 16, 1, unroll=UNR)
      def _(j):
        o = j * 16
        idx = sB[sp][pl.ds(o, 16)]
        vB[m][pl.ds(o, 16)] = plsc.load_gather(u_tile, [idx])

      scats[m] = pltpu.async_copy(vB[m], acc_sh.at[dB[m]], sem_ss[m], add=True)

    for m in range(3):
      if scats[m] is not None:
        scats[m].wait()
    plsc.subcore_barrier()
    pltpu.sync_copy(acc_sh.at[sl], acc_hbm.at[c].at[sl])

  return layer_kernel


def _tc_norms(r):
  def body(dego_ref, degi_ref, h_ref, ns_ref, nd_ref, u0_ref):
    od = dego_ref[0] + dego_ref[1]
    idg = degi_ref[0] + degi_ref[1]
    ns = lax.rsqrt(jnp.maximum(od, 1.0))
    nd = lax.rsqrt(jnp.maximum(idg, 1.0))
    ns_ref[...] = ns
    nd_ref[...] = nd
    u0_ref[...] = h_ref[...] * ns

  return pl.pallas_call(
      body,
      out_shape=[
          jax.ShapeDtypeStruct((r, 128), jnp.float32),
          jax.ShapeDtypeStruct((r, 128), jnp.float32),
          jax.ShapeDtypeStruct((r, 128), jnp.float32),
      ],
  )


def _tc_merge(r):
  def body(acc_ref, ns_ref, nd_ref, u_ref, c5_ref):
    v = (acc_ref[0] + acc_ref[1]) * nd_ref[...]
    u_ref[...] = v * ns_ref[...]
    c5_ref[...] = jnp.sum(v * v).reshape(1, 1)

  return pl.pallas_call(
      body,
      out_shape=[
          jax.ShapeDtypeStruct((r, 128), jnp.float32),
          jax.ShapeDtypeStruct((1, 1), jnp.float32),
      ],
  )


def kernel(h, edge_index):
  n = h.shape[0]
  e = edge_index.shape[1]
  n_pad = ((n + 2047) // 2048) * 2048
  r = n_pad // 128
  e_pad = NW * (((e + NW * CH2 - 1) // (NW * CH2)) * CH2)

  src = edge_index[0].astype(jnp.int32)
  dst = edge_index[1].astype(jnp.int32)
  if e_pad != e:
    pad = jnp.full((e_pad - e,), n_pad - 1, dtype=jnp.int32)
    src = jnp.concatenate([src, pad])
    dst = jnp.concatenate([dst, pad])
  hv = jnp.pad(h[:, 0], (0, n_pad - n))
  zeros = jnp.zeros((n_pad,), jnp.float32)
  ones = jnp.ones((CH,), jnp.float32)

  dego, degi = _sc_degrees(n_pad, e_pad)(src, dst, zeros, ones)
  ns, nd, u = _tc_norms(r)(
      dego.reshape(NC, r, 128), degi.reshape(NC, r, 128), hv.reshape(r, 128)
  )

  layer = _sc_layer(n_pad, e_pad)
  merge = _tc_merge(r)
  c5s = []
  for _ in range(3):
    accp = layer(u.reshape(-1), src, dst, zeros)
    u, c5 = merge(accp.reshape(NC, r, 128), ns, nd)
    c5s.append(c5[0, 0])
  return jnp.stack(c5s)


# speedup vs baseline: 1.9995x; 1.0014x over previous
import functools

import jax
import jax.numpy as jnp
from jax import lax
from jax.experimental import pallas as pl
from jax.experimental.pallas import tpu as pltpu
from jax.experimental.pallas import tpu_sc as plsc

NC = 2
NS = 16
NW = NC * NS
CH = 5888
CH2 = 2944

_mesh = plsc.VectorSubcoreMesh(core_axis_name="c", subcore_axis_name="s")


def _sc_degrees(n_pad, e_pad):
  nch = e_pad // (NW * CH)
  ew = e_pad // NW
  sl_sz = n_pad // NS

  @functools.partial(
      pl.kernel,
      mesh=_mesh,
      out_type=[
          jax.ShapeDtypeStruct((NC, n_pad), jnp.float32),
          jax.ShapeDtypeStruct((NC, n_pad), jnp.float32),
      ],
      scratch_types=[
          [pltpu.VMEM((CH,), jnp.int32)] * 3,
          [pltpu.VMEM((CH,), jnp.int32)] * 3,
          pltpu.VMEM((CH,), jnp.float32),
          pltpu.VMEM_SHARED((n_pad,), jnp.float32),
          pltpu.VMEM_SHARED((n_pad,), jnp.float32),
          [pltpu.SemaphoreType.DMA] * 3,
          [pltpu.SemaphoreType.DMA] * 3,
          [pltpu.SemaphoreType.DMA] * 3,
          [pltpu.SemaphoreType.DMA] * 3,
      ],
  )
  def deg_kernel(src_hbm, dst_hbm, zeros_hbm, ones_hbm,
                 dego_hbm, degi_hbm,
                 sB, dB, ones_buf, dego_sh, degi_sh,
                 sem_ls, sem_ld, sem_so, sem_si):
    c = lax.axis_index("c")
    s = lax.axis_index("s")
    sl = pl.ds(s * sl_sz, sl_sz)
    base = (c * NS + s) * ew

    loads = [None] * 3
    dloads = [None] * 3
    so = [None] * 3
    si = [None] * 3
    loads[0] = pltpu.async_copy(
        src_hbm.at[pl.ds(base, CH)], sB[0], sem_ls[0])
    dloads[0] = pltpu.async_copy(
        dst_hbm.at[pl.ds(base, CH)], dB[0], sem_ld[0])
    pltpu.sync_copy(zeros_hbm.at[sl], dego_sh.at[sl])
    pltpu.sync_copy(zeros_hbm.at[sl], degi_sh.at[sl])
    pltpu.sync_copy(ones_hbm, ones_buf)
    plsc.subcore_barrier()

    for k in range(nch):
      m, mn = k % 3, (k + 1) % 3
      off = base + k * CH
      loads[m].wait()
      dloads[m].wait()
      if k + 1 < nch:
        if so[mn] is not None:
          so[mn].wait()
          so[mn] = None
        if si[mn] is not None:
          si[mn].wait()
          si[mn] = None
        loads[mn] = pltpu.async_copy(
            src_hbm.at[pl.ds(off + CH, CH)], sB[mn], sem_ls[mn])
        dloads[mn] = pltpu.async_copy(
            dst_hbm.at[pl.ds(off + CH, CH)], dB[mn], sem_ld[mn])
      so[m] = pltpu.async_copy(
          ones_buf, dego_sh.at[sB[m]], sem_so[m], add=True)
      si[m] = pltpu.async_copy(
          ones_buf, degi_sh.at[dB[m]], sem_si[m], add=True)

    for m in range(3):
      if so[m] is not None:
        so[m].wait()
      if si[m] is not None:
        si[m].wait()
    plsc.subcore_barrier()
    pltpu.sync_copy(dego_sh.at[sl], dego_hbm.at[c].at[sl])
    pltpu.sync_copy(degi_sh.at[sl], degi_hbm.at[c].at[sl])

  return deg_kernel


def _sc_layer(n_pad, e_pad):
  nch = e_pad // (NW * CH2)
  ew = e_pad // NW
  sl_sz = n_pad // NS
  UNR = 8

  @functools.partial(
      pl.kernel,
      mesh=_mesh,
      out_type=jax.ShapeDtypeStruct((NC, n_pad), jnp.float32),
      scratch_types=[
          pltpu.VMEM((n_pad,), jnp.float32),
          [pltpu.VMEM((CH2,), jnp.int32)] * 2,
          [pltpu.VMEM((CH2,), jnp.int32)] * 3,
          [pltpu.VMEM((CH2,), jnp.float32)] * 3,
          pltpu.VMEM_SHARED((n_pad,), jnp.float32),
          [pltpu.SemaphoreType.DMA] * 2,
          [pltpu.SemaphoreType.DMA] * 3,
          [pltpu.SemaphoreType.DMA] * 3,
      ],
      compiler_params=pltpu.CompilerParams(needs_layout_passes=False),
  )
  def layer_kernel(u_hbm, src_hbm, dst_hbm, zeros_hbm,
                   acc_hbm,
                   u_tile, sB, dB, vB, acc_sh, sem_ls, sem_ld, sem_ss):
    c = lax.axis_index("c")
    s = lax.axis_index("s")
    sl = pl.ds(s * sl_sz, sl_sz)
    base = (c * NS + s) * ew

    loads = [None, None]
    dloads = [None, None, None]
    scats = [None, None, None]
    loads[0] = pltpu.async_copy(
        src_hbm.at[pl.ds(base, CH2)], sB[0], sem_ls[0])
    dloads[0] = pltpu.async_copy(
        dst_hbm.at[pl.ds(base, CH2)], dB[0], sem_ld[0])
    pltpu.sync_copy(zeros_hbm.at[sl], acc_sh.at[sl])
    pltpu.sync_copy(u_hbm, u_tile)
    plsc.subcore_barrier()

    for k in range(nch):
      sp, m = k % 2, k % 3
      spn, mn = (k + 1) % 2, (k + 1) % 3
      off = base + k * CH2
      loads[sp].wait()
      dloads[m].wait()

      if k + 1 < nch:
        if scats[mn] is not None:
          scats[mn].wait()
          scats[mn] = None
        loads[spn] = pltpu.async_copy(
            src_hbm.at[pl.ds(off + CH2, CH2)], sB[spn], sem_ls[spn])
        dloads[mn] = pltpu.async_copy(
            dst_hbm.at[pl.ds(off + CH2, CH2)], dB[mn], sem_ld[mn])

      @plsc.parallel_loop(0, CH2 // 16, 1, unroll=UNR)
      def _(j):
        o = j * 16
        idx = sB[sp][pl.ds(o, 16)]
        vB[m][pl.ds(o, 16)] = plsc.load_gather(u_tile, [idx])

      scats[m] = pltpu.async_copy(vB[m], acc_sh.at[dB[m]], sem_ss[m], add=True)

    for m in range(3):
      if scats[m] is not None:
        scats[m].wait()
    plsc.subcore_barrier()
    pltpu.sync_copy(acc_sh.at[sl], acc_hbm.at[c].at[sl])

  return layer_kernel


def _tc_norms(r):
  def body(dego_ref, degi_ref, h_ref, ns_ref, nd_ref, u0_ref):
    od = dego_ref[0] + dego_ref[1]
    idg = degi_ref[0] + degi_ref[1]
    ns = lax.rsqrt(jnp.maximum(od, 1.0))
    nd = lax.rsqrt(jnp.maximum(idg, 1.0))
    ns_ref[...] = ns
    nd_ref[...] = nd
    u0_ref[...] = h_ref[...] * ns

  return pl.pallas_call(
      body,
      out_shape=[
          jax.ShapeDtypeStruct((r, 128), jnp.float32),
          jax.ShapeDtypeStruct((r, 128), jnp.float32),
          jax.ShapeDtypeStruct((r, 128), jnp.float32),
      ],
  )


def _tc_merge(r):
  def body(acc_ref, ns_ref, nd_ref, u_ref, c5_ref):
    v = (acc_ref[0] + acc_ref[1]) * nd_ref[...]
    u_ref[...] = v * ns_ref[...]
    c5_ref[...] = jnp.sum(v * v).reshape(1, 1)

  return pl.pallas_call(
      body,
      out_shape=[
          jax.ShapeDtypeStruct((r, 128), jnp.float32),
          jax.ShapeDtypeStruct((1, 1), jnp.float32),
      ],
  )


def kernel(h, edge_index):
  n = h.shape[0]
  e = edge_index.shape[1]
  n_pad = ((n + 2047) // 2048) * 2048
  r = n_pad // 128
  e_pad = NW * (((e + NW * CH2 - 1) // (NW * CH2)) * CH2)

  src = edge_index[0].astype(jnp.int32)
  dst = edge_index[1].astype(jnp.int32)
  if e_pad != e:
    pad = jnp.full((e_pad - e,), n_pad - 1, dtype=jnp.int32)
    src = jnp.concatenate([src, pad])
    dst = jnp.concatenate([dst, pad])
  hv = jnp.pad(h[:, 0], (0, n_pad - n))
  zeros = jnp.zeros((n_pad,), jnp.float32)
  ones = jnp.ones((CH,), jnp.float32)

  dego, degi = _sc_degrees(n_pad, e_pad)(src, dst, zeros, ones)
  ns, nd, u = _tc_norms(r)(
      dego.reshape(NC, r, 128), degi.reshape(NC, r, 128), hv.reshape(r, 128)
  )

  layer = _sc_layer(n_pad, e_pad)
  merge = _tc_merge(r)
  c5s = []
  for _ in range(3):
    accp = layer(u.reshape(-1), src, dst, zeros)
    u, c5 = merge(accp.reshape(NC, r, 128), ns, nd)
    c5s.append(c5[0, 0])
  return jnp.stack(c5s)


# speedup vs baseline: 1.9997x; 1.0001x over previous
import functools

import jax
import jax.numpy as jnp
from jax import lax
from jax.experimental import pallas as pl
from jax.experimental.pallas import tpu as pltpu
from jax.experimental.pallas import tpu_sc as plsc

NC = 2
NS = 16
NW = NC * NS
CH = 5888
CH2 = 2944

_mesh = plsc.VectorSubcoreMesh(core_axis_name="c", subcore_axis_name="s")


def _sc_degrees(n_pad, e_pad):
  nch = e_pad // (NW * CH)
  ew = e_pad // NW
  sl_sz = n_pad // NS

  @functools.partial(
      pl.kernel,
      mesh=_mesh,
      out_type=[
          jax.ShapeDtypeStruct((NC, n_pad), jnp.float32),
          jax.ShapeDtypeStruct((NC, n_pad), jnp.float32),
      ],
      scratch_types=[
          [pltpu.VMEM((CH,), jnp.int32)] * 3,
          [pltpu.VMEM((CH,), jnp.int32)] * 3,
          pltpu.VMEM((CH,), jnp.float32),
          pltpu.VMEM_SHARED((n_pad,), jnp.float32),
          pltpu.VMEM_SHARED((n_pad,), jnp.float32),
          [pltpu.SemaphoreType.DMA] * 3,
          [pltpu.SemaphoreType.DMA] * 3,
          [pltpu.SemaphoreType.DMA] * 3,
          [pltpu.SemaphoreType.DMA] * 3,
      ],
  )
  def deg_kernel(src_hbm, dst_hbm, zeros_hbm, ones_hbm,
                 dego_hbm, degi_hbm,
                 sB, dB, ones_buf, dego_sh, degi_sh,
                 sem_ls, sem_ld, sem_so, sem_si):
    c = lax.axis_index("c")
    s = lax.axis_index("s")
    sl = pl.ds(s * sl_sz, sl_sz)
    base = (c * NS + s) * ew

    loads = [None] * 3
    dloads = [None] * 3
    so = [None] * 3
    si = [None] * 3
    loads[0] = pltpu.async_copy(
        src_hbm.at[pl.ds(base, CH)], sB[0], sem_ls[0])
    dloads[0] = pltpu.async_copy(
        dst_hbm.at[pl.ds(base, CH)], dB[0], sem_ld[0])
    pltpu.sync_copy(zeros_hbm.at[sl], dego_sh.at[sl])
    pltpu.sync_copy(zeros_hbm.at[sl], degi_sh.at[sl])
    pltpu.sync_copy(ones_hbm, ones_buf)
    plsc.subcore_barrier()

    for k in range(nch):
      m, mn = k % 3, (k + 1) % 3
      off = base + k * CH
      loads[m].wait()
      dloads[m].wait()
      if k + 1 < nch:
        if so[mn] is not None:
          so[mn].wait()
          so[mn] = None
        if si[mn] is not None:
          si[mn].wait()
          si[mn] = None
        loads[mn] = pltpu.async_copy(
            src_hbm.at[pl.ds(off + CH, CH)], sB[mn], sem_ls[mn])
        dloads[mn] = pltpu.async_copy(
            dst_hbm.at[pl.ds(off + CH, CH)], dB[mn], sem_ld[mn])
      so[m] = pltpu.async_copy(
          ones_buf, dego_sh.at[sB[m]], sem_so[m], add=True)
      si[m] = pltpu.async_copy(
          ones_buf, degi_sh.at[dB[m]], sem_si[m], add=True)

    for m in range(3):
      if so[m] is not None:
        so[m].wait()
      if si[m] is not None:
        si[m].wait()
    plsc.subcore_barrier()
    pltpu.sync_copy(dego_sh.at[sl], dego_hbm.at[c].at[sl])
    pltpu.sync_copy(degi_sh.at[sl], degi_hbm.at[c].at[sl])

  return deg_kernel


def _sc_layer(n_pad, e_pad):
  nch = e_pad // (NW * CH2)
  ew = e_pad // NW
  sl_sz = n_pad // NS
  UNR = 8

  @functools.partial(
      pl.kernel,
      mesh=_mesh,
      out_type=jax.ShapeDtypeStruct((NC, n_pad), jnp.float32),
      scratch_types=[
          pltpu.VMEM((n_pad,), jnp.float32),
          [pltpu.VMEM((CH2,), jnp.int32)] * 2,
          [pltpu.VMEM((CH2,), jnp.int32)] * 3,
          [pltpu.VMEM((CH2,), jnp.float32)] * 3,
          pltpu.VMEM_SHARED((n_pad,), jnp.float32),
          [pltpu.SemaphoreType.DMA] * 2,
          [pltpu.SemaphoreType.DMA] * 3,
          [pltpu.SemaphoreType.DMA] * 3,
      ],
      compiler_params=pltpu.CompilerParams(needs_layout_passes=False),
  )
  def layer_kernel(u_hbm, src_hbm, dst_hbm, zeros_hbm,
                   acc_hbm,
                   u_tile, sB, dB, vB, acc_sh, sem_ls, sem_ld, sem_ss):
    c = lax.axis_index("c")
    s = lax.axis_index("s")
    sl = pl.ds(s * sl_sz, sl_sz)
    base = (c * NS + s) * ew

    loads = [None, None]
    dloads = [None, None, None]
    scats = [None, None, None]
    loads[0] = pltpu.async_copy(
        src_hbm.at[pl.ds(base, CH2)], sB[0], sem_ls[0])
    dloads[0] = pltpu.async_copy(
        dst_hbm.at[pl.ds(base, CH2)], dB[0], sem_ld[0])
    pltpu.sync_copy(zeros_hbm.at[sl], acc_sh.at[sl])
    pltpu.sync_copy(u_hbm, u_tile)
    plsc.subcore_barrier()

    for k in range(nch):
      sp, m = k % 2, k % 3
      spn, mn = (k + 1) % 2, (k + 1) % 3
      off = base + k * CH2
      loads[sp].wait()
      dloads[m].wait()

      if k + 1 < nch:
        if scats[mn] is not None:
          scats[mn].wait()
          scats[mn] = None
        loads[spn] = pltpu.async_copy(
            src_hbm.at[pl.ds(off + CH2, CH2)], sB[spn], sem_ls[spn])
        dloads[mn] = pltpu.async_copy(
            dst_hbm.at[pl.ds(off + CH2, CH2)], dB[mn], sem_ld[mn])

      @plsc.parallel_loop(0, CH2 // 16, 1, unroll=UNR)
      def _(j):
        o = j * 16
        idx = sB[sp][pl.ds(o, 16)]
        vB[m][pl.ds(o, 16)] = plsc.load_gather(u_tile, [idx])

      scats[m] = pltpu.async_copy(vB[m], acc_sh.at[dB[m]], sem_ss[m], add=True)

    for m in range(3):
      if scats[m] is not None:
        scats[m].wait()
    plsc.subcore_barrier()
    pltpu.sync_copy(acc_sh.at[sl], acc_hbm.at[c].at[sl])

  return layer_kernel


def _tc_norms(r):
  def body(dego_ref, degi_ref, h_ref, ns_ref, nd_ref, u0_ref):
    od = dego_ref[0] + dego_ref[1]
    idg = degi_ref[0] + degi_ref[1]
    ns = lax.rsqrt(jnp.maximum(od, 1.0))
    nd = lax.rsqrt(jnp.maximum(idg, 1.0))
    ns_ref[...] = ns
    nd_ref[...] = nd
    u0_ref[...] = h_ref[...] * ns

  return pl.pallas_call(
      body,
      out_shape=[
          jax.ShapeDtypeStruct((r, 128), jnp.float32),
          jax.ShapeDtypeStruct((r, 128), jnp.float32),
          jax.ShapeDtypeStruct((r, 128), jnp.float32),
      ],
  )


def _tc_merge(r):
  def body(acc_ref, ns_ref, nd_ref, u_ref, c5_ref):
    v = (acc_ref[0] + acc_ref[1]) * nd_ref[...]
    u_ref[...] = v * ns_ref[...]
    c5_ref[...] = jnp.sum(v * v).reshape(1, 1)

  return pl.pallas_call(
      body,
      out_shape=[
          jax.ShapeDtypeStruct((r, 128), jnp.float32),
          jax.ShapeDtypeStruct((1, 1), jnp.float32),
      ],
  )


def kernel(h, edge_index):
  n = h.shape[0]
  e = edge_index.shape[1]
  n_pad = ((n + 2047) // 2048) * 2048
  r = n_pad // 128
  e_pad = NW * (((e + NW * CH - 1) // (NW * CH)) * CH)

  src = edge_index[0].astype(jnp.int32)
  dst = edge_index[1].astype(jnp.int32)
  if e_pad != e:
    pad = jnp.full((e_pad - e,), n_pad - 1, dtype=jnp.int32)
    src = jnp.concatenate([src, pad])
    dst = jnp.concatenate([dst, pad])
  hv = jnp.pad(h[:, 0], (0, n_pad - n))
  zeros = jnp.zeros((n_pad,), jnp.float32)
  ones = jnp.ones((CH,), jnp.float32)

  dego, degi = _sc_degrees(n_pad, e_pad)(src, dst, zeros, ones)
  ns, nd, u = _tc_norms(r)(
      dego.reshape(NC, r, 128), degi.reshape(NC, r, 128), hv.reshape(r, 128)
  )

  layer = _sc_layer(n_pad, e_pad)
  merge = _tc_merge(r)
  c5s = []
  for _ in range(3):
    accp = layer(u.reshape(-1), src, dst, zeros)
    u, c5 = merge(accp.reshape(NC, r, 128), ns, nd)
    c5s.append(c5[0, 0])
  return jnp.stack(c5s)
